# Initial kernel scaffold; baseline (speedup 1.0000x reference)
import jax
import jax.numpy as jnp
from jax.experimental import pallas as pl


def kernel(x, eis, W1, b1, W2, b2, Wih, Whh, bih, bhh):
    raise NotImplementedError("write your pallas kernel here")


# speedup vs baseline: 2.2018x; 2.2018x over previous
import functools

import jax
import jax.numpy as jnp
from jax.experimental import pallas as pl
from jax.experimental.pallas import tpu as pltpu

N = 10000
E = 320000
T = 4
D = 128
G = 192
EMB = 64


def _mm_bias_kernel(x_ref, w_ref, b_ref, o_ref):
    o_ref[...] = (
        jnp.dot(x_ref[...], w_ref[...], preferred_element_type=jnp.float32)
        + b_ref[...]
    )


def _matmul_bias(x, w, b, blk=2000):
    n, k = x.shape
    m = w.shape[1]
    return pl.pallas_call(
        _mm_bias_kernel,
        grid=(n // blk,),
        in_specs=[
            pl.BlockSpec((blk, k), lambda i: (i, 0)),
            pl.BlockSpec((k, m), lambda i: (0, 0)),
            pl.BlockSpec((1, m), lambda i: (0, 0)),
        ],
        out_specs=pl.BlockSpec((blk, m), lambda i: (i, 0)),
        out_shape=jax.ShapeDtypeStruct((n, m), jnp.float32),
    )(x, w, b.reshape(1, m))


_GRU_BLK = 250


def _gru_kernel(gi_ref, whhT_ref, bhh_ref, y_ref, h_ref):
    pid = pl.program_id(0)

    @pl.when(pid == 0)
    def _():
        h_ref[...] = jnp.zeros_like(h_ref)

    whhT = whhT_ref[...]
    bhh = bhh_ref[...]

    def sigmoid(v):
        return 1.0 / (1.0 + jnp.exp(-v))

    def body(i, h):
        gi = gi_ref[i]
        gh = jnp.dot(h, whhT, preferred_element_type=jnp.float32) + bhh
        r = sigmoid(gi[:, :EMB] + gh[:, :EMB])
        z = sigmoid(gi[:, EMB : 2 * EMB] + gh[:, EMB : 2 * EMB])
        nn = jnp.tanh(gi[:, 2 * EMB :] + r * gh[:, 2 * EMB :])
        hnew = (1.0 - z) * nn + z * h
        y_ref[i] = hnew
        return hnew

    h_ref[...] = jax.lax.fori_loop(0, _GRU_BLK, body, h_ref[...])


def _gru(gi, whhT, bhh):
    return pl.pallas_call(
        _gru_kernel,
        grid=(N // _GRU_BLK,),
        in_specs=[
            pl.BlockSpec((_GRU_BLK, T, G), lambda i: (i, 0, 0)),
            pl.BlockSpec((EMB, G), lambda i: (0, 0)),
            pl.BlockSpec((1, G), lambda i: (0, 0)),
        ],
        out_specs=pl.BlockSpec((_GRU_BLK, T, EMB), lambda i: (i, 0, 0)),
        out_shape=jax.ShapeDtypeStruct((N, T, EMB), jnp.float32),
        scratch_shapes=[pltpu.VMEM((T, EMB), jnp.float32)],
    )(gi, whhT, bhh.reshape(1, G))


def kernel(x, eis, W1, b1, W2, b2, Wih, Whh, bih, bhh):
    h1 = _matmul_bias(x, W1, jnp.zeros((D,), jnp.float32))

    gis = []
    for t in range(T):
        src = eis[t, 0]
        dst = eis[t, 1]
        deg = jnp.ones((N,), jnp.float32).at[dst].add(1.0)
        dinv = jax.lax.rsqrt(deg)
        ideg = 1.0 / deg

        hs1 = h1 * dinv[:, None]
        acc1 = jnp.zeros((N, D), jnp.float32).at[dst].add(
            hs1[src] * dinv[dst][:, None]
        )
        z1 = jax.nn.relu(acc1 + h1 * ideg[:, None] + b1)

        h2 = _matmul_bias(z1, W2, jnp.zeros((D,), jnp.float32))
        hs2 = h2 * dinv[:, None]
        acc2 = jnp.zeros((N, D), jnp.float32).at[dst].add(
            hs2[src] * dinv[dst][:, None]
        )
        z2 = jax.nn.relu(acc2 + h2 * ideg[:, None] + b2)

        gis.append(_matmul_bias(z2, Wih.T, bih))

    gi = jnp.transpose(jnp.stack(gis), (1, 0, 2))
    ys = _gru(gi, Whh.T, bhh)
    return jnp.transpose(ys, (1, 0, 2))
